# Initial kernel scaffold; baseline (speedup 1.0000x reference)
import jax
import jax.numpy as jnp
from jax.experimental import pallas as pl


def kernel(query_embedding, projector, W, b):
    raise NotImplementedError("write your pallas kernel here")


# speedup vs baseline: 10.0157x; 10.0157x over previous
import jax
import jax.numpy as jnp
from jax.experimental import pallas as pl


def _fused_kernel(q_ref, proj_ref, w_ref, b_ref, out_ref):
    wv = w_ref[0, :]
    m = jnp.sum(proj_ref[:] * wv[:, None, None], axis=0)
    q = q_ref[:]
    acc = jax.lax.dot_general(
        q, m, (((1,), (0,)), ((), ())),
        preferred_element_type=jnp.float32,
    )
    out_ref[:] = acc + b_ref[0, :][None, :]


def kernel(query_embedding, projector, W, b):
    B, D = query_embedding.shape
    P = projector.shape[0]
    BLK = 2048
    grid = (B // BLK,)
    out = pl.pallas_call(
        _fused_kernel,
        grid=grid,
        in_specs=[
            pl.BlockSpec((BLK, D), lambda i: (i, 0)),
            pl.BlockSpec((P, D, D), lambda i: (0, 0, 0)),
            pl.BlockSpec((1, P), lambda i: (0, 0)),
            pl.BlockSpec((1, D), lambda i: (0, 0)),
        ],
        out_specs=pl.BlockSpec((BLK, D), lambda i: (i, 0)),
        out_shape=jax.ShapeDtypeStruct((B, D), jnp.float32),
    )(query_embedding, projector, W, b)
    return out[:, None, :]


# speedup vs baseline: 12.1155x; 1.2097x over previous
import jax
import jax.numpy as jnp
from jax.experimental import pallas as pl


def _fused_kernel(q_ref, proj_ref, w_ref, b_ref, out_ref):
    wv = w_ref[0, :]
    m = jnp.sum(proj_ref[:] * wv[:, None, None], axis=0)
    q = q_ref[:]
    acc = jax.lax.dot_general(
        q, m, (((1,), (0,)), ((), ())),
        preferred_element_type=jnp.float32,
    )
    out_ref[:] = acc + b_ref[0, :][None, :]


def kernel(query_embedding, projector, W, b):
    B, D = query_embedding.shape
    P = projector.shape[0]
    BLK = 8192
    grid = (B // BLK,)
    out = pl.pallas_call(
        _fused_kernel,
        grid=grid,
        in_specs=[
            pl.BlockSpec((BLK, D), lambda i: (i, 0)),
            pl.BlockSpec((P, D, D), lambda i: (0, 0, 0)),
            pl.BlockSpec((1, P), lambda i: (0, 0)),
            pl.BlockSpec((1, D), lambda i: (0, 0)),
        ],
        out_specs=pl.BlockSpec((BLK, D), lambda i: (i, 0)),
        out_shape=jax.ShapeDtypeStruct((B, D), jnp.float32),
    )(query_embedding, projector, W, b)
    return out[:, None, :]
